# baseline (speedup 1.0000x reference)
import functools

import jax
import jax.numpy as jnp
from jax import lax
from jax.experimental import pallas as pl
from jax.experimental.pallas import tpu as pltpu
from jax.experimental.pallas import tpu_sc as plsc

GRID_LO = 32
GRID_HI = 128
F = 4
C = 8
K = 9
OD = 2
BS = 32
P_LO = GRID_LO * GRID_LO
P_HI = GRID_HI * GRID_HI
KDF = K * (OD + 1) * F

PT = 256
NPT = P_HI // PT
NG = PT // 16

_info = plsc.get_sparse_core_info()
_NC = _info.num_cores
_NS = _info.num_subcores


def _body(x_r, orog_r, w_r, bl_r, bh_r, bo_r, cls_r, nbr_r, out_r,
          cls_v, ylow_v, blw_v, v0_v, v1_v, tmp_v,
          w0_v, w1_v, nbr0_v, nbr1_v, bh0_v, bh1_v, o0_v, o1_v,
          sin0, sin1, sout0, sout1):
    b = lax.axis_index("s") * _NC + lax.axis_index("c")

    lane = lax.iota(jnp.int32, 16)
    zero = jnp.zeros((16,), jnp.int32)

    pltpu.sync_copy(cls_r, cls_v)
    csel = (jnp.where(lane == b, cls_v[pl.ds(0, 16)], zero)
            + jnp.where(lane == b - 16, cls_v[pl.ds(16, 16)], zero))
    c = jnp.sum(csel)

    bufs = ((w0_v, nbr0_v, bh0_v, o0_v, sin0, sout0),
            (w1_v, nbr1_v, bh1_v, o1_v, sin1, sout1))

    def start_in(t, ph):
        w_v, nbr_v, bh_v, _, sin, _ = bufs[ph]
        pbase = t * PT
        pltpu.async_copy(w_r.at[c, :, :, :, pl.ds(pbase, PT)], w_v, sin)
        pltpu.async_copy(nbr_r.at[:, pl.ds(pbase, PT)], nbr_v, sin)
        pltpu.async_copy(bh_r.at[c, :, pl.ds(pbase, PT)], bh_v, sin)

    def wait_in(ph):
        w_v, nbr_v, bh_v, _, sin, _ = bufs[ph]
        pltpu.make_async_copy(w_r.at[c, :, :, :, pl.ds(0, PT)], w_v, sin).wait()
        pltpu.make_async_copy(nbr_r.at[:, pl.ds(0, PT)], nbr_v, sin).wait()
        pltpu.make_async_copy(bh_r.at[c, :, pl.ds(0, PT)], bh_v, sin).wait()

    start_in(0, 0)
    start_in(1, 1)

    pltpu.sync_copy(x_r.at[b], ylow_v)
    pltpu.sync_copy(bl_r.at[c], blw_v)
    for f in range(F):

        @pl.loop(0, P_LO // 16)
        def _sub_low(i):
            s = pl.ds(f * P_LO + i * 16, 16)
            ylow_v[s] = ylow_v[s] - blw_v[f, pl.ds(i * 16, 16)]

    for d, v_v in ((0, v0_v), (1, v1_v)):
        pltpu.sync_copy(orog_r.at[b, pl.ds(d * P_HI, P_HI)], v_v)
        pltpu.sync_copy(bo_r.at[c, d], tmp_v)

        @pl.loop(0, P_HI // 16)
        def _sub_o(i):
            s = pl.ds(i * 16, 16)
            v_v[s] = v_v[s] - tmp_v[s]

    @pl.loop(0, NPT // 2)
    def _tile2(tt):
        for ph in range(2):
            w_v, nbr_v, bh_v, o_v, _, sout = bufs[ph]
            t = tt * 2 + ph
            wait_in(ph)
            @pl.when(t >= 2)
            def _drain_out():
                pltpu.make_async_copy(
                    o_v, out_r.at[b, :, pl.ds(0, PT)], sout).wait()

            @pl.loop(0, NG, unroll=2)
            def _grp(g):
                base = g * 16
                acc = [bh_v[f, pl.ds(base, 16)] for f in range(F)]
                for k in range(K):
                    nbr = nbr_v[k, pl.ds(base, 16)]
                    ci = ((nbr >> 9) << 5) | ((nbr >> 2) & 31)
                    a0 = plsc.load_gather(v0_v, [nbr])
                    a1 = plsc.load_gather(v1_v, [nbr])
                    for f in range(F):
                        u = plsc.load_gather(ylow_v, [ci + f * P_LO])
                        w0 = w_v[k, 0, f, pl.ds(base, 16)]
                        w1 = w_v[k, 1, f, pl.ds(base, 16)]
                        w2 = w_v[k, 2, f, pl.ds(base, 16)]
                        acc[f] = acc[f] + u * w0 + a0 * w1 + a1 * w2
                for f in range(F):
                    o_v[f, pl.ds(base, 16)] = acc[f]

            pltpu.async_copy(o_v, out_r.at[b, :, pl.ds(t * PT, PT)], sout)
            start_in(jnp.minimum(t + 2, NPT - 1), ph)

    wait_in(0)
    wait_in(1)
    for ph in range(2):
        _, _, _, o_v, _, sout = bufs[ph]
        pltpu.make_async_copy(o_v, out_r.at[b, :, pl.ds(0, PT)], sout).wait()


@jax.jit
def _run(x, orog_t, wt, bias_low, bh, bo, cls, nbr_t):
    mesh = plsc.VectorSubcoreMesh(core_axis_name="c", subcore_axis_name="s")
    kfn = functools.partial(
        pl.kernel,
        out_type=jax.ShapeDtypeStruct((BS, F, P_HI), jnp.float32),
        mesh=mesh,
        compiler_params=pltpu.CompilerParams(needs_layout_passes=False),
        scratch_types=[
            pltpu.VMEM((BS,), jnp.int32),
            pltpu.VMEM((F * P_LO,), jnp.float32),
            pltpu.VMEM((F, P_LO), jnp.float32),
            pltpu.VMEM((P_HI,), jnp.float32),
            pltpu.VMEM((P_HI,), jnp.float32),
            pltpu.VMEM((P_HI,), jnp.float32),
            pltpu.VMEM((K, OD + 1, F, PT), jnp.float32),
            pltpu.VMEM((K, OD + 1, F, PT), jnp.float32),
            pltpu.VMEM((K, PT), jnp.int32),
            pltpu.VMEM((K, PT), jnp.int32),
            pltpu.VMEM((F, PT), jnp.float32),
            pltpu.VMEM((F, PT), jnp.float32),
            pltpu.VMEM((F, PT), jnp.float32),
            pltpu.VMEM((F, PT), jnp.float32),
            pltpu.SemaphoreType.DMA,
            pltpu.SemaphoreType.DMA,
            pltpu.SemaphoreType.DMA,
            pltpu.SemaphoreType.DMA,
        ],
    )(_body)
    return kfn(x, orog_t, wt, bias_low, bh, bo, cls, nbr_t)


def kernel(x, orog, weight_map, bias_low, bias_high, bias_orog, cls_ids,
           neighbor_indices):
    wt = jnp.transpose(weight_map, (0, 3, 4, 1, 2))
    orog_t = jnp.transpose(orog, (0, 2, 1)).reshape(BS, OD * P_HI)
    nbr_t = jnp.transpose(neighbor_indices.astype(jnp.int32), (1, 0))
    cls = cls_ids.astype(jnp.int32)
    out = _run(x, orog_t, wt, bias_low, bias_high, bias_orog, cls, nbr_t)
    return out.reshape(BS, F, GRID_HI, GRID_HI)


# speedup vs baseline: 1.0103x; 1.0103x over previous
import functools

import jax
import jax.numpy as jnp
from jax import lax
from jax.experimental import pallas as pl
from jax.experimental.pallas import tpu as pltpu
from jax.experimental.pallas import tpu_sc as plsc

GRID_LO = 32
GRID_HI = 128
F = 4
C = 8
K = 9
OD = 2
BS = 32
P_LO = GRID_LO * GRID_LO
P_HI = GRID_HI * GRID_HI
KDF = K * (OD + 1) * F

PT = 256
NPT = P_HI // PT
NG = PT // 16

_info = plsc.get_sparse_core_info()
_NC = _info.num_cores
_NS = _info.num_subcores


def _body(x_r, orog_r, w_r, bl_r, bh_r, bo_r, cls_r, nbr_r, out_r,
          cls_v, ylow_v, blw_v, v0_v, v1_v, tmp_v,
          w0_v, w1_v, nbr0_v, nbr1_v, bh0_v, bh1_v, o0_v, o1_v,
          sin0, sin1, sout0, sout1):
    b = lax.axis_index("s") * _NC + lax.axis_index("c")

    lane = lax.iota(jnp.int32, 16)
    zero = jnp.zeros((16,), jnp.int32)

    pltpu.sync_copy(cls_r, cls_v)
    csel = (jnp.where(lane == b, cls_v[pl.ds(0, 16)], zero)
            + jnp.where(lane == b - 16, cls_v[pl.ds(16, 16)], zero))
    c = jnp.sum(csel)

    bufs = ((w0_v, nbr0_v, bh0_v, o0_v, sin0, sout0),
            (w1_v, nbr1_v, bh1_v, o1_v, sin1, sout1))

    def start_in(t, ph):
        w_v, nbr_v, bh_v, _, sin, _ = bufs[ph]
        pbase = t * PT
        pltpu.async_copy(w_r.at[c, :, :, :, pl.ds(pbase, PT)], w_v, sin)
        pltpu.async_copy(nbr_r.at[:, pl.ds(pbase, PT)], nbr_v, sin)
        pltpu.async_copy(bh_r.at[c, :, pl.ds(pbase, PT)], bh_v, sin)

    def wait_in(ph):
        w_v, nbr_v, bh_v, _, sin, _ = bufs[ph]
        pltpu.make_async_copy(w_r.at[c, :, :, :, pl.ds(0, PT)], w_v, sin).wait()
        pltpu.make_async_copy(nbr_r.at[:, pl.ds(0, PT)], nbr_v, sin).wait()
        pltpu.make_async_copy(bh_r.at[c, :, pl.ds(0, PT)], bh_v, sin).wait()

    start_in(0, 0)
    start_in(1, 1)

    pltpu.sync_copy(x_r.at[b], ylow_v)
    pltpu.sync_copy(bl_r.at[c], blw_v)
    for f in range(F):

        @pl.loop(0, P_LO // 16)
        def _sub_low(i):
            s = pl.ds(f * P_LO + i * 16, 16)
            ylow_v[s] = ylow_v[s] - blw_v[f, pl.ds(i * 16, 16)]

    for d, v_v in ((0, v0_v), (1, v1_v)):
        pltpu.sync_copy(orog_r.at[b, pl.ds(d * P_HI, P_HI)], v_v)
        pltpu.sync_copy(bo_r.at[c, d], tmp_v)

        @pl.loop(0, P_HI // 16)
        def _sub_o(i):
            s = pl.ds(i * 16, 16)
            v_v[s] = v_v[s] - tmp_v[s]

    @pl.loop(0, NPT // 2)
    def _tile2(tt):
        for ph in range(2):
            w_v, nbr_v, bh_v, o_v, _, sout = bufs[ph]
            t = tt * 2 + ph
            wait_in(ph)
            @pl.when(t >= 2)
            def _drain_out():
                pltpu.make_async_copy(
                    o_v, out_r.at[b, :, pl.ds(0, PT)], sout).wait()

            @pl.loop(0, NG, unroll=2)
            def _grp(g):
                base = g * 16
                i_row = 2 * t + (g >> 3)
                j0 = (g & 7) * 16
                r0 = jnp.clip((i_row - 2) >> 2, 0, GRID_LO - 2)
                c0 = jnp.clip((j0 - 2) >> 2, 0, GRID_LO // 2)
                cb = r0 * GRID_LO + c0
                rows = [(ylow_v[pl.ds(f * P_LO + cb, 16)],
                         ylow_v[pl.ds(f * P_LO + cb + GRID_LO, 16)])
                        for f in range(F)]
                acc = [bh_v[f, pl.ds(base, 16)] for f in range(F)]
                for k in range(K):
                    nbr = nbr_v[k, pl.ds(base, 16)]
                    ci = ((nbr >> 9) << 5) | ((nbr >> 2) & 31)
                    loc = ci - cb
                    hi_m = loc >= GRID_LO
                    idx = loc & 15
                    a0 = plsc.load_gather(v0_v, [nbr])
                    a1 = plsc.load_gather(v1_v, [nbr])
                    for f in range(F):
                        lo_r, hi_r = rows[f]
                        u = jnp.where(hi_m,
                                      hi_r.at[idx].get(mode="promise_in_bounds"),
                                      lo_r.at[idx].get(mode="promise_in_bounds"))
                        w0 = w_v[k, 0, f, pl.ds(base, 16)]
                        w1 = w_v[k, 1, f, pl.ds(base, 16)]
                        w2 = w_v[k, 2, f, pl.ds(base, 16)]
                        acc[f] = acc[f] + u * w0 + a0 * w1 + a1 * w2
                for f in range(F):
                    o_v[f, pl.ds(base, 16)] = acc[f]

            pltpu.async_copy(o_v, out_r.at[b, :, pl.ds(t * PT, PT)], sout)
            start_in(jnp.minimum(t + 2, NPT - 1), ph)

    wait_in(0)
    wait_in(1)
    for ph in range(2):
        _, _, _, o_v, _, sout = bufs[ph]
        pltpu.make_async_copy(o_v, out_r.at[b, :, pl.ds(0, PT)], sout).wait()


@jax.jit
def _run(x, orog_t, wt, bias_low, bh, bo, cls, nbr_t):
    mesh = plsc.VectorSubcoreMesh(core_axis_name="c", subcore_axis_name="s")
    kfn = functools.partial(
        pl.kernel,
        out_type=jax.ShapeDtypeStruct((BS, F, P_HI), jnp.float32),
        mesh=mesh,
        compiler_params=pltpu.CompilerParams(needs_layout_passes=False),
        scratch_types=[
            pltpu.VMEM((BS,), jnp.int32),
            pltpu.VMEM((F * P_LO,), jnp.float32),
            pltpu.VMEM((F, P_LO), jnp.float32),
            pltpu.VMEM((P_HI,), jnp.float32),
            pltpu.VMEM((P_HI,), jnp.float32),
            pltpu.VMEM((P_HI,), jnp.float32),
            pltpu.VMEM((K, OD + 1, F, PT), jnp.float32),
            pltpu.VMEM((K, OD + 1, F, PT), jnp.float32),
            pltpu.VMEM((K, PT), jnp.int32),
            pltpu.VMEM((K, PT), jnp.int32),
            pltpu.VMEM((F, PT), jnp.float32),
            pltpu.VMEM((F, PT), jnp.float32),
            pltpu.VMEM((F, PT), jnp.float32),
            pltpu.VMEM((F, PT), jnp.float32),
            pltpu.SemaphoreType.DMA,
            pltpu.SemaphoreType.DMA,
            pltpu.SemaphoreType.DMA,
            pltpu.SemaphoreType.DMA,
        ],
    )(_body)
    return kfn(x, orog_t, wt, bias_low, bh, bo, cls, nbr_t)


def kernel(x, orog, weight_map, bias_low, bias_high, bias_orog, cls_ids,
           neighbor_indices):
    wt = jnp.transpose(weight_map, (0, 3, 4, 1, 2))
    orog_t = jnp.transpose(orog, (0, 2, 1)).reshape(BS, OD * P_HI)
    nbr_t = jnp.transpose(neighbor_indices.astype(jnp.int32), (1, 0))
    cls = cls_ids.astype(jnp.int32)
    out = _run(x, orog_t, wt, bias_low, bias_high, bias_orog, cls, nbr_t)
    return out.reshape(BS, F, GRID_HI, GRID_HI)
